# baseline (speedup 1.0000x reference)
import jax
import jax.numpy as jnp
from jax import lax
from jax.experimental import pallas as pl
from jax.experimental.pallas import tpu as pltpu
from jax.experimental.pallas import tpu_sc as plsc

VOCAB = 1000000
BLOCK = 200
EMBED = 64
B = 4096
L = 200
R = B * L
C = 128
CPL = B // C
ITEMS = R // C
G = 16

NC, NS = 2, 16
NW = NC * NS
IPT = ITEMS // NW


def _sc_body(emb_hbm, xt_hbm, pos_hbm, out_hbm,
             idx_v, idxh_v, rows_v, tblk_v, pos_v, gsem, osem):
    wid = lax.axis_index("s") * NC + lax.axis_index("c")
    base = wid * IPT
    pltpu.sync_copy(pos_hbm, pos_v)
    pltpu.sync_copy(xt_hbm.at[pl.ds(base, IPT)], idx_v)
    iota = lax.iota(jnp.int32, G)

    def fire(t, s):
        for g in range(C // G):
            tv = idx_v[t, pl.ds(g * G, G)]
            idxh_v[s, pl.ds(g * G, G)] = lax.shift_right_logical(tv, 1)
        pltpu.async_copy(emb_hbm.at[idxh_v.at[s]], rows_v.at[s], gsem[s])

    def gwait(s):
        pltpu.make_async_copy(
            emb_hbm.at[idxh_v.at[s]], rows_v.at[s], gsem[s]).wait()

    def out_desc(t, s):
        i = base + t
        l = i // CPL
        b0 = (i % CPL) * C
        return pltpu.make_async_copy(
            tblk_v.at[s], out_hbm.at[l, :, pl.ds(b0, C)], osem[s])

    def consume(t, s):
        i = base + t
        l = i // CPL
        gwait(s)

        @pl.when(t > 1)
        def _():
            out_desc(t - 2, s).wait()

        l_splat = jnp.full((G,), l, jnp.int32)
        s_splat = jnp.full((G,), s, jnp.int32)
        rowv = [jnp.full((G,), g * G, jnp.int32) + iota for g in range(C // G)]
        colb = []
        for g in range(C // G):
            tv = idx_v[t, pl.ds(g * G, G)]
            colb.append(jnp.bitwise_and(tv, 1) * EMBED)

        @plsc.parallel_loop(0, EMBED, 1, unroll=2)
        def _(d):
            d_splat = jnp.full((G,), d, jnp.int32)
            pd = plsc.load_gather(pos_v, [l_splat, d_splat])
            for g in range(C // G):
                v = plsc.load_gather(rows_v, [s_splat, rowv[g], colb[g] + d])
                tblk_v[s, d, pl.ds(g * G, G)] = v + pd

        out_desc(t, s).start()

    fire(0, 0)

    @pl.loop(0, IPT, step=2)
    def _(t):
        fire(t + 1, 1)
        consume(t, 0)

        @pl.when(t + 2 < IPT)
        def _():
            fire(t + 2, 0)

        consume(t + 1, 1)

    out_desc(IPT - 2, 0).wait()
    out_desc(IPT - 1, 1).wait()


@jax.jit
def _sc_lookup(emb2, xt, pos_table):
    mesh = plsc.VectorSubcoreMesh(core_axis_name="c", subcore_axis_name="s")
    return pl.kernel(
        _sc_body,
        out_type=jax.ShapeDtypeStruct((L, EMBED, B), jnp.float32),
        mesh=mesh,
        compiler_params=pltpu.CompilerParams(
            use_tc_tiling_on_sc=True, needs_layout_passes=False),
        scratch_types=[
            pltpu.VMEM((IPT, C), jnp.int32),
            pltpu.VMEM((2, C), jnp.int32),
            pltpu.VMEM((2, C, 2 * EMBED), jnp.float32),
            pltpu.VMEM((2, EMBED, C), jnp.float32),
            pltpu.VMEM((BLOCK, EMBED), jnp.float32),
            [pltpu.SemaphoreType.DMA, pltpu.SemaphoreType.DMA],
            [pltpu.SemaphoreType.DMA, pltpu.SemaphoreType.DMA],
        ],
    )(emb2, xt, pos_table)


def kernel(x, emb_table, pos_table):
    xt = x.T.reshape(R // C, C)
    emb2 = emb_table.reshape(VOCAB // 2, 2 * EMBED)
    out = _sc_lookup(emb2, xt, pos_table)
    return jnp.transpose(out, (0, 2, 1))


# speedup vs baseline: 1.3318x; 1.3318x over previous
import jax
import jax.numpy as jnp
from jax import lax
from jax.experimental import pallas as pl
from jax.experimental.pallas import tpu as pltpu
from jax.experimental.pallas import tpu_sc as plsc

VOCAB = 1000000
BLOCK = 200
EMBED = 64
B = 4096
L = 200
R = B * L
C = 128
CPL = B // C
ITEMS = R // C
PADW = 2 * EMBED

NC, NS = 2, 16
NW = NC * NS
IPT = ITEMS // NW


def _sc_body(emb_hbm, xt_hbm, pos_hbm, out_hbm, idx_v, rows_v, pos_v,
             gsem, osem):
    wid = lax.axis_index("s") * NC + lax.axis_index("c")
    base = wid * IPT
    pltpu.sync_copy(pos_hbm, pos_v)
    pltpu.sync_copy(xt_hbm.at[pl.ds(base, IPT)], idx_v)

    def fire(t, s):
        pltpu.async_copy(emb_hbm.at[idx_v.at[t]], rows_v.at[s], gsem[s])

    def gwait(t, s):
        pltpu.make_async_copy(
            emb_hbm.at[idx_v.at[t]], rows_v.at[s], gsem[s]).wait()

    def out_desc(t, s):
        i = base + t
        l = i // CPL
        b0 = (i % CPL) * C
        return pltpu.make_async_copy(
            rows_v.at[s], out_hbm.at[l, pl.ds(b0, C)], osem[s])

    def consume(t, s):
        i = base + t
        l = i // CPL
        gwait(t, s)

        @pl.when(t > 1)
        def _():
            out_desc(t - 2, s).wait()

        p = [pos_v[l, pl.ds(k * 16, 16)] for k in range(EMBED // 16)]

        @plsc.parallel_loop(0, C, 1, unroll=4)
        def _(j):
            for k in range(EMBED // 16):
                rows_v[s, j, pl.ds(k * 16, 16)] += p[k]

        out_desc(t, s).start()

    fire(0, 0)

    @pl.loop(0, IPT, step=2)
    def _(t):
        fire(t + 1, 1)
        consume(t, 0)

        @pl.when(t + 2 < IPT)
        def _():
            fire(t + 2, 0)

        consume(t + 1, 1)

    out_desc(IPT - 2, 0).wait()
    out_desc(IPT - 1, 1).wait()


@jax.jit
def _sc_lookup(embp, xt, pos_table):
    mesh = plsc.VectorSubcoreMesh(core_axis_name="c", subcore_axis_name="s")
    return pl.kernel(
        _sc_body,
        out_type=jax.ShapeDtypeStruct((L, B, PADW), jnp.float32),
        mesh=mesh,
        compiler_params=pltpu.CompilerParams(
            use_tc_tiling_on_sc=True, needs_layout_passes=False),
        scratch_types=[
            pltpu.VMEM((IPT, C), jnp.int32),
            pltpu.VMEM((2, C, PADW), jnp.float32),
            pltpu.VMEM((BLOCK, EMBED), jnp.float32),
            [pltpu.SemaphoreType.DMA, pltpu.SemaphoreType.DMA],
            [pltpu.SemaphoreType.DMA, pltpu.SemaphoreType.DMA],
        ],
    )(embp, xt, pos_table)


def kernel(x, emb_table, pos_table):
    xt = x.T.reshape(R // C, C)
    embp = jnp.pad(emb_table, ((0, 0), (0, EMBED)))
    out = _sc_lookup(embp, xt, pos_table)
    return out[:, :, :EMBED]
